# baseline (speedup 1.0000x reference)
import jax
import jax.numpy as jnp
from jax import lax
from jax.experimental import pallas as pl
from jax.experimental.pallas import tpu as pltpu
from jax.experimental.pallas import tpu_sc as plsc

B = 16384
D = 768
N_MODALITY = 4
L = 16
NW = 32
B_PER_W = B // NW
CH = 128
NCHUNK = B_PER_W // CH
D_VECS = D // L


def _body(idx_hbm, code_hbm, qtab_hbm, mtab_hbm, out_hbm,
          idx_v, code_v, mod_v, buf, sem):
    wid = lax.axis_index("s") * 2 + lax.axis_index("c")
    wbase = wid * B_PER_W

    pltpu.sync_copy(mtab_hbm, mod_v)

    for c in range(NCHUNK):
        base = wbase + c * CH
        pltpu.sync_copy(idx_hbm.at[pl.ds(base, CH)], idx_v)
        pltpu.sync_copy(code_hbm.at[pl.ds(base, CH)], code_v)
        pltpu.async_copy(qtab_hbm.at[idx_v], buf, sem).wait()

        def grp_body(g, _):
            cv = code_v[pl.ds(g * L, L)] * D

            def col_body(j, _):
                s = j * L
                for k in range(L):
                    i = g * L + k
                    buf[i, pl.ds(s, L)] = (
                        buf[i, pl.ds(s, L)] + mod_v[pl.ds(cv[k] + s, L)]
                    )
                return 0

            lax.fori_loop(0, D_VECS, col_body, 0)
            return 0

        lax.fori_loop(0, CH // L, grp_body, 0)
        pltpu.sync_copy(buf, out_hbm.at[pl.ds(base, CH)])


@jax.jit
def _run(idx, code, qtab, mtab_flat):
    mesh = plsc.VectorSubcoreMesh(core_axis_name="c", subcore_axis_name="s")
    return pl.kernel(
        _body,
        out_type=jax.ShapeDtypeStruct((B, D), jnp.float32),
        mesh=mesh,
        scratch_types=[
            pltpu.VMEM((CH,), jnp.int32),
            pltpu.VMEM((CH,), jnp.int32),
            pltpu.VMEM((N_MODALITY * D,), jnp.float32),
            pltpu.VMEM((CH, D), jnp.float32),
            pltpu.SemaphoreType.DMA,
        ],
    )(idx, code, qtab, mtab_flat)


def kernel(input_text, modality_code, query_table, modality_table):
    idx = input_text.astype(jnp.int32)
    code = modality_code.astype(jnp.int32)
    return _run(idx, code, query_table, modality_table.reshape(-1))
